# Initial kernel scaffold; baseline (speedup 1.0000x reference)
import jax
import jax.numpy as jnp
from jax.experimental import pallas as pl


def kernel(time_vector, linear_trend, constant_offset, residual_amplitudes, residual_phases, residual_periods, emd_spatial_weights, emd_seasonal_components, neighbor_indices, neighbor_weights):
    raise NotImplementedError("write your pallas kernel here")


# speedup vs baseline: 3.6395x; 3.6395x over previous
import functools

import jax
import jax.numpy as jnp
import numpy as np
from jax import lax
from jax.experimental import pallas as pl
from jax.experimental.pallas import tpu as pltpu
from jax.experimental.pallas import tpu_sc as plsc

_SMOOTH = 0.15
_PW = 16
_PTW = 128


def _prep_body(comp_ref, amp_ref, ph_ref, emd_ref, par_ref):
    c = comp_ref[...]
    t = emd_ref.shape[1]
    emd_ref[...] = (c[:, 0 * t:1 * t] + c[:, 1 * t:2 * t]
                    + c[:, 2 * t:3 * t] + c[:, 3 * t:4 * t])
    a = amp_ref[...]
    p = ph_ref[...]
    z = jnp.zeros((a.shape[0], _PTW - 9), jnp.float32)
    par_ref[...] = jnp.concatenate([a, jnp.cos(p), jnp.sin(p), z], axis=1)


def _combine_body(emd_ref, nbe_ref, par_ref, nbp_ref, sc3_ref, basis_ref, out_ref):
    emd = emd_ref[...]
    nbe = nbe_ref[...]
    par = par_ref[...]
    nbp = nbp_ref[...]
    sc3 = sc3_ref[...]
    basis = basis_ref[...]
    mix = jax.nn.sigmoid(sc3[:, 2:3])
    out = (1.0 - mix) * emd + mix * nbe
    out = out + sc3[:, 0:1] * basis[0:1, :] + sc3[:, 1:2] * basis[1:2, :]
    sf = _SMOOTH
    for c in range(3):
        a_s = (1.0 - sf) * par[:, c:c + 1] + sf * nbp[:, c:c + 1]
        re = (1.0 - sf) * par[:, 3 + c:4 + c] + sf * nbp[:, 3 + c:4 + c]
        im = (1.0 - sf) * par[:, 6 + c:7 + c] + sf * nbp[:, 6 + c:7 + c]
        inv = lax.rsqrt(re * re + im * im)
        out = out + (a_s * re * inv) * basis[2 + c:3 + c, :] \
                  + (a_s * im * inv) * basis[5 + c:6 + c, :]
    out_ref[...] = out


def _sc_gather(emd_tab, par_tab, idx_flat, w_flat, npad, n_k):
    info = plsc.get_sparse_core_info()
    nc, ns, lanes = info.num_cores, info.num_subcores, info.num_lanes
    nw = nc * ns
    t = emd_tab.shape[1]
    nv = t // lanes
    spw = npad // nw
    cs = 8
    ech = cs * n_k
    nch = spw // cs
    mesh = plsc.VectorSubcoreMesh(core_axis_name="c", subcore_axis_name="s")

    @functools.partial(
        pl.kernel,
        mesh=mesh,
        out_type=(jax.ShapeDtypeStruct((npad, t), jnp.float32),
                  jax.ShapeDtypeStruct((npad, _PW), jnp.float32)),
        scratch_types=[
            pltpu.VMEM((2, ech), jnp.int32),
            pltpu.VMEM((2, ech), jnp.float32),
            pltpu.VMEM((2, ech, t), jnp.float32),
            pltpu.VMEM((2, ech, _PTW), jnp.float32),
            pltpu.VMEM((cs, t), jnp.float32),
            pltpu.VMEM((cs, _PW), jnp.float32),
            pltpu.SemaphoreType.DMA,
            pltpu.SemaphoreType.DMA,
            pltpu.SemaphoreType.DMA,
            pltpu.SemaphoreType.DMA,
        ],
    )
    def sck(emd_hbm, par_hbm, idx_hbm, w_hbm, oemd_hbm, opar_hbm,
            idxb, wb, rowsb, prowsb, oemd, opar, es0, es1, ps0, ps1):
        esem = (es0, es1)
        psem = (ps0, ps1)
        wid = lax.axis_index("s") * nc + lax.axis_index("c")
        sbase = wid * spw
        ebase = sbase * n_k

        def issue(c, b):
            off = ebase + c * ech
            pltpu.sync_copy(idx_hbm.at[pl.ds(off, ech)], idxb.at[b])
            pltpu.sync_copy(w_hbm.at[pl.ds(off, ech)], wb.at[b])
            pltpu.async_copy(emd_hbm.at[idxb.at[b]], rowsb.at[b], esem[b])
            pltpu.async_copy(par_hbm.at[idxb.at[b]], prowsb.at[b], psem[b])

        def wait(b):
            pltpu.make_async_copy(emd_hbm.at[idxb.at[b]], rowsb.at[b], esem[b]).wait()
            pltpu.make_async_copy(par_hbm.at[idxb.at[b]], prowsb.at[b], psem[b]).wait()

        def compute(c, b):
            def st(s, carry):
                r0 = s * n_k
                wv = wb[b, pl.ds(r0, n_k)]
                w0 = wv[0]
                accs = [w0 * rowsb[b, r0, pl.ds(v * lanes, lanes)] for v in range(nv)]
                pacc = w0 * prowsb[b, r0, pl.ds(0, _PW)]
                for k in range(1, n_k):
                    rr = r0 + k
                    wk = wv[k]
                    for v in range(nv):
                        accs[v] = accs[v] + wk * rowsb[b, rr, pl.ds(v * lanes, lanes)]
                    pacc = pacc + wk * prowsb[b, rr, pl.ds(0, _PW)]
                for v in range(nv):
                    oemd[s, pl.ds(v * lanes, lanes)] = accs[v]
                opar[s, :] = pacc
                return carry
            lax.fori_loop(0, cs, st, 0)
            row = sbase + c * cs
            pltpu.sync_copy(oemd, oemd_hbm.at[pl.ds(row, cs)])
            pltpu.sync_copy(opar, opar_hbm.at[pl.ds(row, cs)])

        issue(0, 0)
        issue(1, 1)

        def pair(j, carry):
            c0 = j * 2
            for b in range(2):
                c = c0 + b
                wait(b)
                compute(c, b)

                @pl.when(c + 2 < nch)
                def _():
                    issue(c + 2, b)
            return carry

        lax.fori_loop(0, nch // 2, pair, 0)

    return sck(emd_tab, par_tab, idx_flat, w_flat)


def kernel(time_vector, linear_trend, constant_offset, residual_amplitudes,
           residual_phases, residual_periods, emd_spatial_weights,
           emd_seasonal_components, neighbor_indices, neighbor_weights):
    n, n_k = neighbor_indices.shape
    t = time_vector.shape[0]
    bn = 1000
    grid = n // bn

    comp2 = emd_seasonal_components.reshape(n, 4 * t)
    emd_tab, par_tab = pl.pallas_call(
        _prep_body,
        grid=(grid,),
        in_specs=[pl.BlockSpec((bn, 4 * t), lambda i: (i, 0)),
                  pl.BlockSpec((bn, 3), lambda i: (i, 0)),
                  pl.BlockSpec((bn, 3), lambda i: (i, 0))],
        out_specs=[pl.BlockSpec((bn, t), lambda i: (i, 0)),
                   pl.BlockSpec((bn, _PTW), lambda i: (i, 0))],
        out_shape=(jax.ShapeDtypeStruct((n, t), jnp.float32),
                   jax.ShapeDtypeStruct((n, _PTW), jnp.float32)),
    )(comp2, residual_amplitudes, residual_phases)

    npad = ((n + 255) // 256) * 256
    idx_flat = jnp.pad(neighbor_indices, ((0, npad - n), (0, 0))).reshape(-1)
    w_flat = jnp.pad(neighbor_weights, ((0, npad - n), (0, 0))).reshape(-1)

    nb_emd, nb_par = _sc_gather(emd_tab, par_tab, idx_flat, w_flat, npad, n_k)

    freq = 1.0 / residual_periods
    ang = (2.0 * np.pi) * freq[:, None] * time_vector[None, :]
    basis = jnp.concatenate([jnp.ones((1, t), jnp.float32), time_vector[None, :],
                             jnp.sin(ang), jnp.cos(ang)], axis=0)
    sc3 = jnp.stack([constant_offset, linear_trend, emd_spatial_weights], axis=1)

    out = pl.pallas_call(
        _combine_body,
        grid=(grid,),
        in_specs=[pl.BlockSpec((bn, t), lambda i: (i, 0)),
                  pl.BlockSpec((bn, t), lambda i: (i, 0)),
                  pl.BlockSpec((bn, _PTW), lambda i: (i, 0)),
                  pl.BlockSpec((bn, _PW), lambda i: (i, 0)),
                  pl.BlockSpec((bn, 3), lambda i: (i, 0)),
                  pl.BlockSpec((8, t), lambda i: (0, 0))],
        out_specs=pl.BlockSpec((bn, t), lambda i: (i, 0)),
        out_shape=jax.ShapeDtypeStruct((n, t), jnp.float32),
    )(emd_tab, nb_emd, par_tab, nb_par, sc3, basis)
    return out


# speedup vs baseline: 6.1329x; 1.6851x over previous
import functools

import jax
import jax.numpy as jnp
import numpy as np
from jax import lax
from jax.experimental import pallas as pl
from jax.experimental.pallas import tpu as pltpu
from jax.experimental.pallas import tpu_sc as plsc

_SMOOTH = 0.15
_PW = 16
_PTW = 128


def _prep_body(comp_ref, amp_ref, ph_ref, emd_ref, par_ref):
    c = comp_ref[...]
    t = emd_ref.shape[1]
    emd_ref[...] = (c[:, 0 * t:1 * t] + c[:, 1 * t:2 * t]
                    + c[:, 2 * t:3 * t] + c[:, 3 * t:4 * t])
    a = amp_ref[...]
    p = ph_ref[...]
    z = jnp.zeros((a.shape[0], _PTW - 9), jnp.float32)
    par_ref[...] = jnp.concatenate([a, jnp.cos(p), jnp.sin(p), z], axis=1)


def _combine_body(emd_ref, nbe_ref, par_ref, nbp_ref, sc3_ref, basis_ref, out_ref):
    emd = emd_ref[...]
    nbe = nbe_ref[...]
    par = par_ref[...]
    nbp = nbp_ref[...]
    sc3 = sc3_ref[...]
    basis = basis_ref[...]
    mix = jax.nn.sigmoid(sc3[:, 2:3])
    out = (1.0 - mix) * emd + mix * nbe
    out = out + sc3[:, 0:1] * basis[0:1, :] + sc3[:, 1:2] * basis[1:2, :]
    sf = _SMOOTH
    for c in range(3):
        a_s = (1.0 - sf) * par[:, c:c + 1] + sf * nbp[:, c:c + 1]
        re = (1.0 - sf) * par[:, 3 + c:4 + c] + sf * nbp[:, 3 + c:4 + c]
        im = (1.0 - sf) * par[:, 6 + c:7 + c] + sf * nbp[:, 6 + c:7 + c]
        inv = lax.rsqrt(re * re + im * im)
        out = out + (a_s * re * inv) * basis[2 + c:3 + c, :] \
                  + (a_s * im * inv) * basis[5 + c:6 + c, :]
    out_ref[...] = out


_CORE0_CHUNKS = 40
_CORE1_CHUNKS = 40


def _sc_gather(emd_tab, par_tab, idx_flat, w_flat, n, n_k):
    info = plsc.get_sparse_core_info()
    nc, ns, lanes = info.num_cores, info.num_subcores, info.num_lanes
    t = emd_tab.shape[1]
    nv = t // lanes
    cs = 8
    ech = cs * n_k
    nch = n // cs
    ca, cb = _CORE0_CHUNKS, _CORE1_CHUNKS
    maxslot = max(ca, cb)
    npairs = (maxslot + 1) // 2
    mesh = plsc.VectorSubcoreMesh(core_axis_name="c", subcore_axis_name="s")

    @functools.partial(
        pl.kernel,
        mesh=mesh,
        out_type=(jax.ShapeDtypeStruct((n, t), jnp.float32),
                  jax.ShapeDtypeStruct((n, _PW), jnp.float32)),
        scratch_types=[
            pltpu.VMEM((2, ech), jnp.int32),
            pltpu.VMEM((2, ech), jnp.float32),
            pltpu.VMEM((2, ech, t), jnp.float32),
            pltpu.VMEM((2, ech, _PTW), jnp.float32),
            pltpu.VMEM((cs, t), jnp.float32),
            pltpu.VMEM((cs, _PW), jnp.float32),
            pltpu.SemaphoreType.DMA,
            pltpu.SemaphoreType.DMA,
            pltpu.SemaphoreType.DMA,
            pltpu.SemaphoreType.DMA,
        ],
    )
    def sck(emd_hbm, par_hbm, idx_hbm, w_hbm, oemd_hbm, opar_hbm,
            idxb, wb, rowsb, prowsb, oemd, opar, es0, es1, ps0, ps1):
        esem = (es0, es1)
        psem = (ps0, ps1)
        cid = lax.axis_index("c")
        sid = lax.axis_index("s")
        start = jnp.where(cid == 0, sid * ca, ns * ca + sid * cb)
        slots = jnp.where(cid == 0, ca, cb)
        cnt = jnp.clip(nch - start, 0, slots)

        def issue(c, b):
            off = (start + c) * ech
            pltpu.sync_copy(idx_hbm.at[pl.ds(off, ech)], idxb.at[b])
            pltpu.sync_copy(w_hbm.at[pl.ds(off, ech)], wb.at[b])
            pltpu.async_copy(emd_hbm.at[idxb.at[b]], rowsb.at[b], esem[b])
            pltpu.async_copy(par_hbm.at[idxb.at[b]], prowsb.at[b], psem[b])

        def wait(b):
            pltpu.make_async_copy(emd_hbm.at[idxb.at[b]], rowsb.at[b], esem[b]).wait()
            pltpu.make_async_copy(par_hbm.at[idxb.at[b]], prowsb.at[b], psem[b]).wait()

        def compute(c, b):
            def st(s, carry):
                r0 = s * n_k
                wv = wb[b, pl.ds(r0, n_k)]
                w0 = wv[0]
                accs = [w0 * rowsb[b, r0, pl.ds(v * lanes, lanes)] for v in range(nv)]
                pacc = w0 * prowsb[b, r0, pl.ds(0, _PW)]
                for k in range(1, n_k):
                    rr = r0 + k
                    wk = wv[k]
                    for v in range(nv):
                        accs[v] = accs[v] + wk * rowsb[b, rr, pl.ds(v * lanes, lanes)]
                    pacc = pacc + wk * prowsb[b, rr, pl.ds(0, _PW)]
                for v in range(nv):
                    oemd[s, pl.ds(v * lanes, lanes)] = accs[v]
                opar[s, :] = pacc
                return carry
            lax.fori_loop(0, cs, st, 0)
            row = (start + c) * cs
            pltpu.sync_copy(oemd, oemd_hbm.at[pl.ds(row, cs)])
            pltpu.sync_copy(opar, opar_hbm.at[pl.ds(row, cs)])

        @pl.when(cnt > 0)
        def _():
            issue(0, 0)

        @pl.when(cnt > 1)
        def _():
            issue(1, 1)

        def pair(j, carry):
            c0 = j * 2
            for b in range(2):
                c = c0 + b

                @pl.when(c < cnt)
                def _():
                    wait(b)
                    compute(c, b)

                @pl.when(c + 2 < cnt)
                def _():
                    issue(c + 2, b)
            return carry

        lax.fori_loop(0, npairs, pair, 0)

    return sck(emd_tab, par_tab, idx_flat, w_flat)


def kernel(time_vector, linear_trend, constant_offset, residual_amplitudes,
           residual_phases, residual_periods, emd_spatial_weights,
           emd_seasonal_components, neighbor_indices, neighbor_weights):
    n, n_k = neighbor_indices.shape
    t = time_vector.shape[0]
    bn = 1000
    grid = n // bn

    comp2 = emd_seasonal_components.reshape(n, 4 * t)
    emd_tab, par_tab = pl.pallas_call(
        _prep_body,
        grid=(grid,),
        in_specs=[pl.BlockSpec((bn, 4 * t), lambda i: (i, 0)),
                  pl.BlockSpec((bn, 3), lambda i: (i, 0)),
                  pl.BlockSpec((bn, 3), lambda i: (i, 0))],
        out_specs=[pl.BlockSpec((bn, t), lambda i: (i, 0)),
                   pl.BlockSpec((bn, _PTW), lambda i: (i, 0))],
        out_shape=(jax.ShapeDtypeStruct((n, t), jnp.float32),
                   jax.ShapeDtypeStruct((n, _PTW), jnp.float32)),
    )(comp2, residual_amplitudes, residual_phases)

    idx_flat = neighbor_indices.reshape(-1)
    w_flat = neighbor_weights.reshape(-1)
    nb_emd, nb_par = _sc_gather(emd_tab, par_tab, idx_flat, w_flat, n, n_k)

    freq = 1.0 / residual_periods
    ang = (2.0 * np.pi) * freq[:, None] * time_vector[None, :]
    basis = jnp.concatenate([jnp.ones((1, t), jnp.float32), time_vector[None, :],
                             jnp.sin(ang), jnp.cos(ang)], axis=0)
    sc3 = jnp.stack([constant_offset, linear_trend, emd_spatial_weights], axis=1)

    out = pl.pallas_call(
        _combine_body,
        grid=(grid,),
        in_specs=[pl.BlockSpec((bn, t), lambda i: (i, 0)),
                  pl.BlockSpec((bn, t), lambda i: (i, 0)),
                  pl.BlockSpec((bn, _PTW), lambda i: (i, 0)),
                  pl.BlockSpec((bn, _PW), lambda i: (i, 0)),
                  pl.BlockSpec((bn, 3), lambda i: (i, 0)),
                  pl.BlockSpec((8, t), lambda i: (0, 0))],
        out_specs=pl.BlockSpec((bn, t), lambda i: (i, 0)),
        out_shape=jax.ShapeDtypeStruct((n, t), jnp.float32),
    )(emd_tab, nb_emd, par_tab, nb_par, sc3, basis)
    return out


# speedup vs baseline: 6.1564x; 1.0038x over previous
import functools

import jax
import jax.numpy as jnp
import numpy as np
from jax import lax
from jax.experimental import pallas as pl
from jax.experimental.pallas import tpu as pltpu
from jax.experimental.pallas import tpu_sc as plsc

_SMOOTH = 0.15
_PW = 16
_PTW = 128


def _prep_body(comp_ref, amp_ref, ph_ref, emd_ref, par_ref):
    c = comp_ref[...]
    t = emd_ref.shape[1]
    emd_ref[...] = (c[:, 0 * t:1 * t] + c[:, 1 * t:2 * t]
                    + c[:, 2 * t:3 * t] + c[:, 3 * t:4 * t])
    a = amp_ref[...]
    p = ph_ref[...]
    z = jnp.zeros((a.shape[0], _PTW - 9), jnp.float32)
    par_ref[...] = jnp.concatenate([a, jnp.cos(p), jnp.sin(p), z], axis=1)


def _combine_body(emd_ref, nbe_ref, par_ref, nbp_ref, sc3_ref, basis_ref, out_ref):
    emd = emd_ref[...]
    nbe = nbe_ref[...]
    par = par_ref[...]
    nbp = nbp_ref[...]
    sc3 = sc3_ref[...]
    basis = basis_ref[...]
    mix = jax.nn.sigmoid(sc3[:, 2:3])
    out = (1.0 - mix) * emd + mix * nbe
    out = out + sc3[:, 0:1] * basis[0:1, :] + sc3[:, 1:2] * basis[1:2, :]
    sf = _SMOOTH
    for c in range(3):
        a_s = (1.0 - sf) * par[:, c:c + 1] + sf * nbp[:, c:c + 1]
        re = (1.0 - sf) * par[:, 3 + c:4 + c] + sf * nbp[:, 3 + c:4 + c]
        im = (1.0 - sf) * par[:, 6 + c:7 + c] + sf * nbp[:, 6 + c:7 + c]
        inv = lax.rsqrt(re * re + im * im)
        out = out + (a_s * re * inv) * basis[2 + c:3 + c, :] \
                  + (a_s * im * inv) * basis[5 + c:6 + c, :]
    out_ref[...] = out


_CORE0_CHUNKS = 40
_CORE1_CHUNKS = 40


def _sc_gather(emd_tab, par_tab, idx_flat, w_flat, n, n_k):
    info = plsc.get_sparse_core_info()
    nc, ns, lanes = info.num_cores, info.num_subcores, info.num_lanes
    t = emd_tab.shape[1]
    nv = t // lanes
    cs = 8
    ech = cs * n_k
    nch = n // cs
    ca, cb = _CORE0_CHUNKS, _CORE1_CHUNKS
    maxslot = max(ca, cb)
    npairs = (maxslot + 1) // 2
    mesh = plsc.VectorSubcoreMesh(core_axis_name="c", subcore_axis_name="s")

    @functools.partial(
        pl.kernel,
        mesh=mesh,
        compiler_params=pltpu.CompilerParams(use_tc_tiling_on_sc=True),
        out_type=(jax.ShapeDtypeStruct((n, t), jnp.float32),
                  jax.ShapeDtypeStruct((n, _PW), jnp.float32)),
        scratch_types=[
            pltpu.VMEM((2, ech), jnp.int32),
            pltpu.VMEM((2, ech), jnp.float32),
            pltpu.VMEM((2, ech, t), jnp.float32),
            pltpu.VMEM((2, ech, _PTW), jnp.float32),
            pltpu.VMEM((cs, t), jnp.float32),
            pltpu.VMEM((cs, _PW), jnp.float32),
            pltpu.SemaphoreType.DMA,
            pltpu.SemaphoreType.DMA,
            pltpu.SemaphoreType.DMA,
            pltpu.SemaphoreType.DMA,
        ],
    )
    def sck(emd_hbm, par_hbm, idx_hbm, w_hbm, oemd_hbm, opar_hbm,
            idxb, wb, rowsb, prowsb, oemd, opar, es0, es1, ps0, ps1):
        esem = (es0, es1)
        psem = (ps0, ps1)
        cid = lax.axis_index("c")
        sid = lax.axis_index("s")
        start = jnp.where(cid == 0, sid * ca, ns * ca + sid * cb)
        slots = jnp.where(cid == 0, ca, cb)
        cnt = jnp.clip(nch - start, 0, slots)

        def issue(c, b):
            off = (start + c) * ech
            pltpu.sync_copy(idx_hbm.at[pl.ds(off, ech)], idxb.at[b])
            pltpu.sync_copy(w_hbm.at[pl.ds(off, ech)], wb.at[b])
            pltpu.async_copy(emd_hbm.at[idxb.at[b]], rowsb.at[b], esem[b])
            pltpu.async_copy(par_hbm.at[idxb.at[b]], prowsb.at[b], psem[b])

        def wait(b):
            pltpu.make_async_copy(emd_hbm.at[idxb.at[b]], rowsb.at[b], esem[b]).wait()
            pltpu.make_async_copy(par_hbm.at[idxb.at[b]], prowsb.at[b], psem[b]).wait()

        def compute(c, b):
            def st(s, carry):
                r0 = s * n_k
                wv = wb[b, pl.ds(r0, n_k)]
                w0 = wv[0]
                accs = [w0 * rowsb[b, r0, pl.ds(v * lanes, lanes)] for v in range(nv)]
                pacc = w0 * prowsb[b, r0, pl.ds(0, _PW)]
                for k in range(1, n_k):
                    rr = r0 + k
                    wk = wv[k]
                    for v in range(nv):
                        accs[v] = accs[v] + wk * rowsb[b, rr, pl.ds(v * lanes, lanes)]
                    pacc = pacc + wk * prowsb[b, rr, pl.ds(0, _PW)]
                for v in range(nv):
                    oemd[s, pl.ds(v * lanes, lanes)] = accs[v]
                opar[s, :] = pacc
                return carry
            lax.fori_loop(0, cs, st, 0)
            row = (start + c) * cs
            pltpu.sync_copy(oemd, oemd_hbm.at[pl.ds(row, cs)])
            pltpu.sync_copy(opar, opar_hbm.at[pl.ds(row, cs)])

        @pl.when(cnt > 0)
        def _():
            issue(0, 0)

        @pl.when(cnt > 1)
        def _():
            issue(1, 1)

        def pair(j, carry):
            c0 = j * 2
            for b in range(2):
                c = c0 + b

                @pl.when(c < cnt)
                def _():
                    wait(b)
                    compute(c, b)

                @pl.when(c + 2 < cnt)
                def _():
                    issue(c + 2, b)
            return carry

        lax.fori_loop(0, npairs, pair, 0)

    return sck(emd_tab, par_tab, idx_flat, w_flat)


def kernel(time_vector, linear_trend, constant_offset, residual_amplitudes,
           residual_phases, residual_periods, emd_spatial_weights,
           emd_seasonal_components, neighbor_indices, neighbor_weights):
    n, n_k = neighbor_indices.shape
    t = time_vector.shape[0]
    bn = 1000
    grid = n // bn

    comp2 = emd_seasonal_components.reshape(n, 4 * t)
    emd_tab, par_tab = pl.pallas_call(
        _prep_body,
        grid=(grid,),
        in_specs=[pl.BlockSpec((bn, 4 * t), lambda i: (i, 0)),
                  pl.BlockSpec((bn, 3), lambda i: (i, 0)),
                  pl.BlockSpec((bn, 3), lambda i: (i, 0))],
        out_specs=[pl.BlockSpec((bn, t), lambda i: (i, 0)),
                   pl.BlockSpec((bn, _PTW), lambda i: (i, 0))],
        out_shape=(jax.ShapeDtypeStruct((n, t), jnp.float32),
                   jax.ShapeDtypeStruct((n, _PTW), jnp.float32)),
    )(comp2, residual_amplitudes, residual_phases)

    idx_flat = neighbor_indices.reshape(-1)
    w_flat = neighbor_weights.reshape(-1)
    nb_emd, nb_par = _sc_gather(emd_tab, par_tab, idx_flat, w_flat, n, n_k)

    freq = 1.0 / residual_periods
    ang = (2.0 * np.pi) * freq[:, None] * time_vector[None, :]
    basis = jnp.concatenate([jnp.ones((1, t), jnp.float32), time_vector[None, :],
                             jnp.sin(ang), jnp.cos(ang)], axis=0)
    sc3 = jnp.stack([constant_offset, linear_trend, emd_spatial_weights], axis=1)

    out = pl.pallas_call(
        _combine_body,
        grid=(grid,),
        in_specs=[pl.BlockSpec((bn, t), lambda i: (i, 0)),
                  pl.BlockSpec((bn, t), lambda i: (i, 0)),
                  pl.BlockSpec((bn, _PTW), lambda i: (i, 0)),
                  pl.BlockSpec((bn, _PW), lambda i: (i, 0)),
                  pl.BlockSpec((bn, 3), lambda i: (i, 0)),
                  pl.BlockSpec((8, t), lambda i: (0, 0))],
        out_specs=pl.BlockSpec((bn, t), lambda i: (i, 0)),
        out_shape=jax.ShapeDtypeStruct((n, t), jnp.float32),
    )(emd_tab, nb_emd, par_tab, nb_par, sc3, basis)
    return out
